# baseline (speedup 1.0000x reference)
import functools

import jax
import jax.numpy as jnp
import numpy as np
from jax import lax
from jax.experimental import pallas as pl
from jax.experimental.pallas import tpu as pltpu
from jax.experimental.pallas import tpu_sc as plsc

VOCAB = 100000
HIDDEN = 1024
BATCH = 4
SEQ = 2048

NC = 2
NS = 16
NW = NC * NS
TOTAL = BATCH * SEQ
ROWS_PER_W = TOTAL // NW
CHUNK = 32
NCHUNK = ROWS_PER_W // CHUNK
LANES = 16


def _pos_encoding() -> np.ndarray:
    pos = np.arange(SEQ)[:, None].astype(np.float64)
    i = np.arange(HIDDEN // 2)[None, :].astype(np.float64)
    angle = pos / np.power(10000.0, 2.0 * i / HIDDEN)
    pe = np.zeros((SEQ, HIDDEN), dtype=np.float64)
    pe[:, 0::2] = np.sin(angle)
    pe[:, 1::2] = np.cos(angle)
    return pe.astype(np.float32)


_PE = _pos_encoding()


def _embed_body(ids_hbm, pe_hbm, table_hbm, out_hbm, idx_v, pe_v, buf, sem):
    c = lax.axis_index("c")
    s = lax.axis_index("s")
    wid = s * NC + c
    base = wid * ROWS_PER_W
    s_base = lax.rem(base, SEQ)

    pltpu.sync_copy(ids_hbm.at[wid], idx_v)

    for ch in range(NCHUNK):
        row0 = base + ch * CHUNK
        gather = pltpu.async_copy(table_hbm.at[idx_v.at[ch]], buf, sem)
        pltpu.sync_copy(pe_hbm.at[pl.ds(s_base + ch * CHUNK, CHUNK)], pe_v)
        gather.wait()

        def add_row(r, _):
            for k in range(HIDDEN // LANES):
                sl = pl.ds(k * LANES, LANES)
                buf[r, sl] = buf[r, sl] + pe_v[r, sl]
            return 0

        lax.fori_loop(0, CHUNK, add_row, 0)

        pltpu.sync_copy(buf, out_hbm.at[pl.ds(row0, CHUNK)])


@jax.jit
def _embed(ids3, pe, table):
    mesh = plsc.VectorSubcoreMesh(core_axis_name="c", subcore_axis_name="s")
    f = pl.kernel(
        _embed_body,
        out_type=jax.ShapeDtypeStruct((TOTAL, HIDDEN), jnp.float32),
        mesh=mesh,
        scratch_types=[
            pltpu.VMEM((NCHUNK, CHUNK), jnp.int32),
            pltpu.VMEM((CHUNK, HIDDEN), jnp.float32),
            pltpu.VMEM((CHUNK, HIDDEN), jnp.float32),
            pltpu.SemaphoreType.DMA,
        ],
    )
    return f(ids3, pe, table)


def kernel(input_ids, token_embed_weight):
    ids3 = input_ids.astype(jnp.int32).reshape(NW, NCHUNK, CHUNK)
    pe = jnp.asarray(_PE)
    out = _embed(ids3, pe, token_embed_weight)
    return out.reshape(BATCH, SEQ, HIDDEN)
